# initial kernel scaffold (unmeasured)
import functools

import jax
import jax.numpy as jnp
from jax import lax
from jax.experimental import pallas as pl
from jax.experimental.pallas import tpu as pltpu

N_DEV = 32
N_HOP = N_DEV - 1


def kernel(x, w_mat):
    m, k_per = x.shape
    _, n = w_mat.shape
    m_per = m // N_DEV

    def body(x_ref, w_ref, out_ref, send_buf, recv_buf, send_sems, recv_sems):
        d = lax.axis_index("i")
        left = (d - 1) % N_DEV
        right = (d + 1) % N_DEV

        def partial_chunk(c):
            xc = x_ref[pl.ds(c * m_per, m_per), :]
            return jnp.dot(xc, w_ref[:, :], preferred_element_type=jnp.float32)

        barrier_sem = pltpu.get_barrier_semaphore()
        for nbr in (left, right):
            pl.semaphore_signal(
                barrier_sem, inc=1,
                device_id=(nbr,), device_id_type=pl.DeviceIdType.MESH,
            )
        pl.semaphore_wait(barrier_sem, 2)

        send_buf[0] = partial_chunk((d - 1) % N_DEV).astype(jnp.bfloat16)

        for s in range(N_HOP):
            rdma = pltpu.make_async_remote_copy(
                src_ref=send_buf.at[s],
                dst_ref=recv_buf.at[s],
                send_sem=send_sems.at[s],
                recv_sem=recv_sems.at[s],
                device_id=(right,),
                device_id_type=pl.DeviceIdType.MESH,
            )
            rdma.start()
            part = partial_chunk((d - s - 2) % N_DEV)
            rdma.wait()
            acc = recv_buf[s].astype(jnp.float32) + part
            if s < N_HOP - 1:
                send_buf[s + 1] = acc.astype(jnp.bfloat16)
            else:
                c = 0.7978845608028654
                out_ref[:, :] = 0.5 * acc * (
                    1.0 + jnp.tanh(c * (acc + 0.044715 * acc * acc * acc))
                )

        @functools.partial(
            pl.run_scoped, second_barrier=pltpu.SemaphoreType.REGULAR
        )
        def _(second_barrier):
            for nbr in (left, right):
                pl.semaphore_signal(
                    second_barrier, inc=1,
                    device_id=(nbr,), device_id_type=pl.DeviceIdType.MESH,
                )
            pl.semaphore_wait(second_barrier, 2)

    return pl.pallas_call(
        body,
        out_shape=jax.ShapeDtypeStruct((m_per, n), jnp.float32),
        in_specs=[
            pl.BlockSpec(memory_space=pltpu.VMEM),
            pl.BlockSpec(memory_space=pltpu.VMEM),
        ],
        out_specs=pl.BlockSpec(memory_space=pltpu.VMEM),
        scratch_shapes=[
            pltpu.VMEM((N_HOP, m_per, n), jnp.bfloat16),
            pltpu.VMEM((N_HOP, m_per, n), jnp.bfloat16),
            pltpu.SemaphoreType.DMA((N_HOP,)),
            pltpu.SemaphoreType.DMA((N_HOP,)),
        ],
        compiler_params=pltpu.CompilerParams(collective_id=0),
    )(x, w_mat)


# baseline (device time: 252655 ns/iter reference)
import functools

import jax
import jax.numpy as jnp
from jax import lax
from jax.experimental import pallas as pl
from jax.experimental.pallas import tpu as pltpu

N_DEV = 32
N_HOP = N_DEV - 1


def kernel(x, w_mat):
    m, k_per = x.shape
    _, n = w_mat.shape
    m_per = m // N_DEV

    def body(x_ref, w_ref, out_ref, w_bf, send_buf, recv_buf, send_sems,
             recv_sems):
        d = lax.axis_index("i")
        left = (d - 1) % N_DEV
        right = (d + 1) % N_DEV

        w_bf[:, :] = w_ref[:, :].astype(jnp.bfloat16)

        def partial_chunk(c):
            xc = x_ref[pl.ds(c * m_per, m_per), :].astype(jnp.bfloat16)
            return jnp.dot(xc, w_bf[:, :], preferred_element_type=jnp.float32)

        barrier_sem = pltpu.get_barrier_semaphore()
        for nbr in (left, right):
            pl.semaphore_signal(
                barrier_sem, inc=1,
                device_id=(nbr,), device_id_type=pl.DeviceIdType.MESH,
            )
        pl.semaphore_wait(barrier_sem, 2)

        send_buf[0] = partial_chunk((d - 1) % N_DEV).astype(jnp.bfloat16)

        for s in range(N_HOP):
            rdma = pltpu.make_async_remote_copy(
                src_ref=send_buf.at[s],
                dst_ref=recv_buf.at[s],
                send_sem=send_sems.at[s],
                recv_sem=recv_sems.at[s],
                device_id=(right,),
                device_id_type=pl.DeviceIdType.MESH,
            )
            rdma.start()
            part = partial_chunk((d - s - 2) % N_DEV)
            rdma.wait()
            acc = recv_buf[s].astype(jnp.float32) + part
            if s < N_HOP - 1:
                send_buf[s + 1] = acc.astype(jnp.bfloat16)
            else:
                c = 0.7978845608028654
                out_ref[:, :] = 0.5 * acc * (
                    1.0 + jnp.tanh(c * (acc + 0.044715 * acc * acc * acc))
                )

        @functools.partial(
            pl.run_scoped, second_barrier=pltpu.SemaphoreType.REGULAR
        )
        def _(second_barrier):
            for nbr in (left, right):
                pl.semaphore_signal(
                    second_barrier, inc=1,
                    device_id=(nbr,), device_id_type=pl.DeviceIdType.MESH,
                )
            pl.semaphore_wait(second_barrier, 2)

    return pl.pallas_call(
        body,
        out_shape=jax.ShapeDtypeStruct((m_per, n), jnp.float32),
        in_specs=[
            pl.BlockSpec(memory_space=pltpu.VMEM),
            pl.BlockSpec(memory_space=pltpu.VMEM),
        ],
        out_specs=pl.BlockSpec(memory_space=pltpu.VMEM),
        scratch_shapes=[
            pltpu.VMEM((k_per, n), jnp.bfloat16),
            pltpu.VMEM((N_HOP, m_per, n), jnp.bfloat16),
            pltpu.VMEM((N_HOP, m_per, n), jnp.bfloat16),
            pltpu.SemaphoreType.DMA((N_HOP,)),
            pltpu.SemaphoreType.DMA((N_HOP,)),
        ],
        compiler_params=pltpu.CompilerParams(
            collective_id=0,
            vmem_limit_bytes=56 * 1024 * 1024,
        ),
    )(x, w_mat)


# device time: 234235 ns/iter; 1.0786x vs baseline; 1.0786x over previous
import functools

import jax
import jax.numpy as jnp
from jax import lax
from jax.experimental import pallas as pl
from jax.experimental.pallas import tpu as pltpu

N_DEV = 32
N_HOP = N_DEV - 1


def kernel(x, w_mat):
    m, k_per = x.shape
    _, n = w_mat.shape
    m_per = m // N_DEV
    nh = n // 2

    def body(x_ref, w_ref, out_ref, w_bf,
             send_p, recv_p, send_m, recv_m,
             ssem_p, rsem_p, ssem_m, rsem_m):
        d = lax.axis_index("i")
        left = (d - 1) % N_DEV
        right = (d + 1) % N_DEV

        w_bf[:, :] = w_ref[:, :].astype(jnp.bfloat16)

        def partial(c, col0):
            xc = x_ref[pl.ds(c * m_per, m_per), :].astype(jnp.bfloat16)
            return jnp.dot(xc, w_bf[:, pl.ds(col0, nh)],
                           preferred_element_type=jnp.float32)

        barrier_sem = pltpu.get_barrier_semaphore()
        for nbr in (left, right):
            pl.semaphore_signal(
                barrier_sem, inc=1,
                device_id=(nbr,), device_id_type=pl.DeviceIdType.MESH,
            )
        pl.semaphore_wait(barrier_sem, 2)

        send_p[0] = partial((d - 1) % N_DEV, 0).astype(jnp.bfloat16)
        send_m[0] = partial((d + 1) % N_DEV, nh).astype(jnp.bfloat16)

        for s in range(N_HOP):
            rdma_p = pltpu.make_async_remote_copy(
                src_ref=send_p.at[s], dst_ref=recv_p.at[s],
                send_sem=ssem_p.at[s], recv_sem=rsem_p.at[s],
                device_id=(right,), device_id_type=pl.DeviceIdType.MESH,
            )
            rdma_m = pltpu.make_async_remote_copy(
                src_ref=send_m.at[s], dst_ref=recv_m.at[s],
                send_sem=ssem_m.at[s], recv_sem=rsem_m.at[s],
                device_id=(left,), device_id_type=pl.DeviceIdType.MESH,
            )
            rdma_p.start()
            rdma_m.start()
            part_p = partial((d - s - 2) % N_DEV, 0)
            part_m = partial((d + s + 2) % N_DEV, nh)
            rdma_p.wait()
            acc_p = recv_p[s].astype(jnp.float32) + part_p
            if s < N_HOP - 1:
                send_p[s + 1] = acc_p.astype(jnp.bfloat16)
            rdma_m.wait()
            acc_m = recv_m[s].astype(jnp.float32) + part_m
            if s < N_HOP - 1:
                send_m[s + 1] = acc_m.astype(jnp.bfloat16)
            else:
                c = 0.7978845608028654
                out_ref[:, pl.ds(0, nh)] = 0.5 * acc_p * (
                    1.0 + jnp.tanh(c * (acc_p + 0.044715 * acc_p * acc_p * acc_p))
                )
                out_ref[:, pl.ds(nh, nh)] = 0.5 * acc_m * (
                    1.0 + jnp.tanh(c * (acc_m + 0.044715 * acc_m * acc_m * acc_m))
                )

        @functools.partial(
            pl.run_scoped, second_barrier=pltpu.SemaphoreType.REGULAR
        )
        def _(second_barrier):
            for nbr in (left, right):
                pl.semaphore_signal(
                    second_barrier, inc=1,
                    device_id=(nbr,), device_id_type=pl.DeviceIdType.MESH,
                )
            pl.semaphore_wait(second_barrier, 2)

    return pl.pallas_call(
        body,
        out_shape=jax.ShapeDtypeStruct((m_per, n), jnp.float32),
        in_specs=[
            pl.BlockSpec(memory_space=pltpu.VMEM),
            pl.BlockSpec(memory_space=pltpu.VMEM),
        ],
        out_specs=pl.BlockSpec(memory_space=pltpu.VMEM),
        scratch_shapes=[
            pltpu.VMEM((k_per, n), jnp.bfloat16),
            pltpu.VMEM((N_HOP, m_per, nh), jnp.bfloat16),
            pltpu.VMEM((N_HOP, m_per, nh), jnp.bfloat16),
            pltpu.VMEM((N_HOP, m_per, nh), jnp.bfloat16),
            pltpu.VMEM((N_HOP, m_per, nh), jnp.bfloat16),
            pltpu.SemaphoreType.DMA((N_HOP,)),
            pltpu.SemaphoreType.DMA((N_HOP,)),
            pltpu.SemaphoreType.DMA((N_HOP,)),
            pltpu.SemaphoreType.DMA((N_HOP,)),
        ],
        compiler_params=pltpu.CompilerParams(
            collective_id=0,
            vmem_limit_bytes=56 * 1024 * 1024,
        ),
    )(x, w_mat)
